# baseline (speedup 1.0000x reference)
import jax
import jax.numpy as jnp
from jax import lax
from jax.experimental import pallas as pl
from jax.experimental.pallas import tpu as pltpu

_K = 9
_G = 16


def _fused_kernel(xt_ref, w1_ref, s1_ref, t1_ref, wd_ref, sd_ref, td_ref,
                  wcat_ref, be_ref, w2_ref, s2_ref, t2_ref, o_ref):
    N = xt_ref.shape[1]
    C = xt_ref.shape[2]
    O2 = be_ref.shape[1]
    coli = lax.broadcasted_iota(jnp.int32, (N, N), 1)
    for g in range(_G):
        xt = xt_ref[g]
        h = jnp.dot(xt, w1_ref[...], preferred_element_type=jnp.float32)
        h = h * s1_ref[...] + t1_ref[...]

        code = jnp.dot(h, wd_ref[...], preferred_element_type=jnp.float32)
        code = jnp.maximum(code * sd_ref[...] + td_ref[...], 0.0)
        nrm = jnp.sqrt(jnp.sum(code * code, axis=-1, keepdims=True))
        code_n = code * (1.0 / jnp.maximum(nrm, 1e-12))

        sim = lax.dot_general(code_n, code_n, (((1,), (1,)), ((), ())),
                              preferred_element_type=jnp.float32)

        cat = jnp.dot(h, wcat_ref[...], preferred_element_type=jnp.float32)
        yb = cat[:, O2:]
        pb = cat[:, :O2] + be_ref[...] - yb

        acc = None
        s = sim
        for k in range(_K):
            sel = jnp.argmax(s, axis=1, keepdims=True)
            pmask = coli == sel
            pf = jnp.where(pmask, 1.0, 0.0)
            contrib = jnp.dot(pf, yb,
                              preferred_element_type=jnp.float32)
            acc = contrib if acc is None else jnp.maximum(acc, contrib)
            if k + 1 < _K:
                s = jnp.where(pmask, -3.0, s)

        gact = jnp.maximum(pb + acc, 0.0)
        out = jnp.dot(gact, w2_ref[...], preferred_element_type=jnp.float32)
        out = out * s2_ref[...] + t2_ref[...]
        o_ref[g] = out[:, :C] + xt


def kernel(x, fc1_w, fc1_b, bn1_g, bn1_b, bn1_m, bn1_v, down_w,
           bnd_g, bnd_b, bnd_m, bnd_v, edge_w, edge_b, fc2_w, fc2_b,
           bn2_g, bn2_b, bn2_m, bn2_v):
    B, C, H, W = x.shape
    N = H * W
    Cd = down_w.shape[1]
    O2 = 2 * C
    Cp = 256
    eps = 1e-5

    s1 = bn1_g / jnp.sqrt(bn1_v + eps)
    t1 = bn1_b - bn1_m * s1
    shift1 = fc1_b * s1 + t1
    sd = bnd_g / jnp.sqrt(bnd_v + eps)
    td = bnd_b - bnd_m * sd
    s2 = bn2_g / jnp.sqrt(bn2_v + eps)
    t2 = bn2_b - bn2_m * s2
    shift2 = fc2_b * s2 + t2

    pad = Cp - C
    w1p = jnp.pad(fc1_w, ((0, 0), (0, pad)))
    s1p = jnp.pad(s1, (0, pad)).reshape(1, Cp)
    t1p = jnp.pad(shift1, (0, pad)).reshape(1, Cp)
    wdp = jnp.pad(down_w, ((0, pad), (0, 0)))
    wcat = jnp.pad(jnp.concatenate([edge_w[:C], edge_w[C:]], axis=1),
                   ((0, pad), (0, 0)))
    w2p = jnp.pad(fc2_w, ((0, 0), (0, pad)))
    s2p = jnp.pad(s2, (0, pad)).reshape(1, Cp)
    t2p = jnp.pad(shift2, (0, pad)).reshape(1, Cp)
    sdr = sd.reshape(1, Cd)
    tdr = td.reshape(1, Cd)
    ber = edge_b.reshape(1, O2)

    xt = jnp.transpose(x, (0, 2, 3, 1)).reshape(B, N, C)

    full2 = lambda b: (0, 0)
    bmap3 = lambda b: (b, 0, 0)
    out_t = pl.pallas_call(
        _fused_kernel,
        out_shape=jax.ShapeDtypeStruct((B, N, C), jnp.float32),
        grid=(B // _G,),
        in_specs=[
            pl.BlockSpec((_G, N, C), bmap3),
            pl.BlockSpec((C, Cp), full2),
            pl.BlockSpec((1, Cp), full2),
            pl.BlockSpec((1, Cp), full2),
            pl.BlockSpec((Cp, Cd), full2),
            pl.BlockSpec((1, Cd), full2),
            pl.BlockSpec((1, Cd), full2),
            pl.BlockSpec((Cp, 2 * O2), full2),
            pl.BlockSpec((1, O2), full2),
            pl.BlockSpec((O2, Cp), full2),
            pl.BlockSpec((1, Cp), full2),
            pl.BlockSpec((1, Cp), full2),
        ],
        out_specs=pl.BlockSpec((_G, N, C), bmap3),
        compiler_params=pltpu.CompilerParams(
            dimension_semantics=("parallel",)),
    )(xt, w1p, s1p, t1p, wdp, sdr, tdr, wcat, ber, w2p, s2p, t2p)

    out = out_t.reshape(B, H, W, C).transpose(0, 3, 1, 2)
    return out, jnp.float32(0.0)


# speedup vs baseline: 1.4000x; 1.4000x over previous
import jax
import jax.numpy as jnp
from jax import lax
from jax.experimental import pallas as pl
from jax.experimental.pallas import tpu as pltpu

_K = 9
_G = 16


def _fused_kernel(xt_ref, w1_ref, s1_ref, t1_ref, wd_ref, sd_ref, td_ref,
                  wcat_ref, be_ref, w2_ref, s2_ref, t2_ref, o_ref):
    N = xt_ref.shape[1]
    C = xt_ref.shape[2]
    O2 = be_ref.shape[1]
    rowi = lax.broadcasted_iota(jnp.int32, (N, N), 0)
    for g in range(_G):
        xt = xt_ref[g]
        h = jnp.dot(xt, w1_ref[...], preferred_element_type=jnp.float32)
        h = h * s1_ref[...] + t1_ref[...]

        code = jnp.dot(h, wd_ref[...], preferred_element_type=jnp.float32)
        code = jnp.maximum(code * sd_ref[...] + td_ref[...], 0.0)
        nrm = jnp.sqrt(jnp.sum(code * code, axis=-1, keepdims=True))
        code_n = code * (1.0 / jnp.maximum(nrm, 1e-12))

        sim = lax.dot_general(code_n, code_n, (((1,), (1,)), ((), ())),
                              preferred_element_type=jnp.float32)

        cat = jnp.dot(h, wcat_ref[...], preferred_element_type=jnp.float32)
        yb = cat[:, O2:]
        pb = cat[:, :O2] + be_ref[...] - yb

        contribs = []
        s = sim
        for k in range(_K):
            sel = jnp.argmax(s, axis=0, keepdims=True)
            pmask = rowi == sel
            pf = jnp.where(pmask, 1.0, 0.0)
            contribs.append(
                lax.dot_general(pf, yb, (((0,), (0,)), ((), ())),
                                preferred_element_type=jnp.float32))
            if k + 1 < _K:
                s = jnp.where(pmask, -3.0, s)
        while len(contribs) > 1:
            contribs = [jnp.maximum(*contribs[i:i + 2])
                        if i + 1 < len(contribs) else contribs[i]
                        for i in range(0, len(contribs), 2)]
        acc = contribs[0]

        gact = jnp.maximum(pb + acc, 0.0)
        out = jnp.dot(gact, w2_ref[...], preferred_element_type=jnp.float32)
        out = out * s2_ref[...] + t2_ref[...]
        o_ref[g] = out[:, :C] + xt


def kernel(x, fc1_w, fc1_b, bn1_g, bn1_b, bn1_m, bn1_v, down_w,
           bnd_g, bnd_b, bnd_m, bnd_v, edge_w, edge_b, fc2_w, fc2_b,
           bn2_g, bn2_b, bn2_m, bn2_v):
    B, C, H, W = x.shape
    N = H * W
    Cd = down_w.shape[1]
    O2 = 2 * C
    Cp = 256
    eps = 1e-5

    s1 = bn1_g / jnp.sqrt(bn1_v + eps)
    t1 = bn1_b - bn1_m * s1
    shift1 = fc1_b * s1 + t1
    sd = bnd_g / jnp.sqrt(bnd_v + eps)
    td = bnd_b - bnd_m * sd
    s2 = bn2_g / jnp.sqrt(bn2_v + eps)
    t2 = bn2_b - bn2_m * s2
    shift2 = fc2_b * s2 + t2

    pad = Cp - C
    w1p = jnp.pad(fc1_w, ((0, 0), (0, pad)))
    s1p = jnp.pad(s1, (0, pad)).reshape(1, Cp)
    t1p = jnp.pad(shift1, (0, pad)).reshape(1, Cp)
    wdp = jnp.pad(down_w, ((0, pad), (0, 0)))
    wcat = jnp.pad(jnp.concatenate([edge_w[:C], edge_w[C:]], axis=1),
                   ((0, pad), (0, 0)))
    w2p = jnp.pad(fc2_w, ((0, 0), (0, pad)))
    s2p = jnp.pad(s2, (0, pad)).reshape(1, Cp)
    t2p = jnp.pad(shift2, (0, pad)).reshape(1, Cp)
    sdr = sd.reshape(1, Cd)
    tdr = td.reshape(1, Cd)
    ber = edge_b.reshape(1, O2)

    xt = jnp.transpose(x, (0, 2, 3, 1)).reshape(B, N, C)

    full2 = lambda b: (0, 0)
    bmap3 = lambda b: (b, 0, 0)
    out_t = pl.pallas_call(
        _fused_kernel,
        out_shape=jax.ShapeDtypeStruct((B, N, C), jnp.float32),
        grid=(B // _G,),
        in_specs=[
            pl.BlockSpec((_G, N, C), bmap3),
            pl.BlockSpec((C, Cp), full2),
            pl.BlockSpec((1, Cp), full2),
            pl.BlockSpec((1, Cp), full2),
            pl.BlockSpec((Cp, Cd), full2),
            pl.BlockSpec((1, Cd), full2),
            pl.BlockSpec((1, Cd), full2),
            pl.BlockSpec((Cp, 2 * O2), full2),
            pl.BlockSpec((1, O2), full2),
            pl.BlockSpec((O2, Cp), full2),
            pl.BlockSpec((1, Cp), full2),
            pl.BlockSpec((1, Cp), full2),
        ],
        out_specs=pl.BlockSpec((_G, N, C), bmap3),
        compiler_params=pltpu.CompilerParams(
            dimension_semantics=("parallel",)),
    )(xt, w1p, s1p, t1p, wdp, sdr, tdr, wcat, ber, w2p, s2p, t2p)

    out = out_t.reshape(B, H, W, C).transpose(0, 3, 1, 2)
    return out, jnp.float32(0.0)
